# baseline (speedup 1.0000x reference)
import functools

import jax
import jax.numpy as jnp
from jax import lax
from jax.experimental import pallas as pl
from jax.experimental.pallas import tpu as pltpu
from jax.experimental.pallas import tpu_sc as plsc

NUM_SC_CORES = 2
NUM_SUBCORES = 16
NUM_WORKERS = NUM_SC_CORES * NUM_SUBCORES

BATCH = 16384
EMBED = 32
ROWS_PER_WORKER = BATCH // NUM_WORKERS
IDX_CHUNK = 128
NUM_CHUNKS = ROWS_PER_WORKER // IDX_CHUNK

TC_BLK = 2048


def _sc_gather(user_table, item_table, uid3, iid3):
    mesh = plsc.VectorSubcoreMesh(core_axis_name="c", subcore_axis_name="s")

    @functools.partial(
        pl.kernel,
        mesh=mesh,
        compiler_params=pltpu.CompilerParams(use_tc_tiling_on_sc=False),
        out_type=(
            jax.ShapeDtypeStruct((BATCH, EMBED), jnp.float32),
            jax.ShapeDtypeStruct((BATCH, EMBED), jnp.float32),
        ),
        scratch_types=[
            pltpu.VMEM((NUM_CHUNKS, IDX_CHUNK), jnp.int32),
            pltpu.VMEM((NUM_CHUNKS, IDX_CHUNK), jnp.int32),
            pltpu.VMEM((ROWS_PER_WORKER, EMBED), jnp.float32),
            pltpu.VMEM((ROWS_PER_WORKER, EMBED), jnp.float32),
            pltpu.SemaphoreType.DMA,
            pltpu.SemaphoreType.DMA,
        ],
    )
    def k(ut_hbm, it_hbm, uid_hbm, iid_hbm, u_out, i_out,
          uidx_v, iidx_v, urows_v, irows_v, sem_u, sem_i):
        wid = lax.axis_index("s") * NUM_SC_CORES + lax.axis_index("c")
        base = wid * ROWS_PER_WORKER
        pltpu.sync_copy(uid_hbm.at[wid], uidx_v)
        pltpu.sync_copy(iid_hbm.at[wid], iidx_v)
        copies = []
        for j in range(NUM_CHUNKS):
            dst = urows_v.at[pl.ds(j * IDX_CHUNK, IDX_CHUNK)]
            copies.append(pltpu.async_copy(ut_hbm.at[uidx_v.at[j]], dst, sem_u))
        for j in range(NUM_CHUNKS):
            dst = irows_v.at[pl.ds(j * IDX_CHUNK, IDX_CHUNK)]
            copies.append(pltpu.async_copy(it_hbm.at[iidx_v.at[j]], dst, sem_i))
        for c in copies:
            c.wait()
        pltpu.sync_copy(urows_v, u_out.at[pl.ds(base, ROWS_PER_WORKER)])
        pltpu.sync_copy(irows_v, i_out.at[pl.ds(base, ROWS_PER_WORKER)])

    return k(user_table, item_table, uid3, iid3)


def _tc_body(u_ref, i_ref, f_ref, wf_ref, bf_ref, w_ref, b_ref, out_ref):
    g = lax.dot_general(f_ref[...], wf_ref[...], (((1,), (1,)), ((), ())),
                        preferred_element_type=jnp.float32)
    g = g + bf_ref[...]
    inter = u_ref[...] * (i_ref[...] + g)
    out_ref[...] = (jnp.sum(inter * w_ref[...], axis=1) + b_ref[0, 0])[None, None, :]


def _tc_combine(u, i, features, Wf, bf2, W, b2):
    nf = features.shape[1]
    nblk = BATCH // TC_BLK
    return pl.pallas_call(
        _tc_body,
        grid=(nblk,),
        in_specs=[
            pl.BlockSpec((TC_BLK, EMBED), lambda idx: (idx, 0)),
            pl.BlockSpec((TC_BLK, EMBED), lambda idx: (idx, 0)),
            pl.BlockSpec((TC_BLK, nf), lambda idx: (idx, 0)),
            pl.BlockSpec((EMBED, nf), lambda idx: (0, 0)),
            pl.BlockSpec((1, EMBED), lambda idx: (0, 0)),
            pl.BlockSpec((1, EMBED), lambda idx: (0, 0)),
            pl.BlockSpec((1, 1), lambda idx: (0, 0)),
        ],
        out_specs=pl.BlockSpec((1, 1, TC_BLK), lambda idx: (idx, 0, 0)),
        out_shape=jax.ShapeDtypeStruct((nblk, 1, TC_BLK), jnp.float32),
    )(u, i, features, Wf, bf2, W, b2)


def kernel(user_ids, item_ids, features, user_table, item_table, Wf, bf, W, b):
    uid3 = user_ids.astype(jnp.int32).reshape(NUM_WORKERS, NUM_CHUNKS, IDX_CHUNK)
    iid3 = item_ids.astype(jnp.int32).reshape(NUM_WORKERS, NUM_CHUNKS, IDX_CHUNK)
    u, i = _sc_gather(user_table, item_table, uid3, iid3)
    out2d = _tc_combine(u, i, features, Wf, bf.reshape(1, EMBED), W,
                        b.reshape(1, 1))
    return out2d.reshape(BATCH)


# speedup vs baseline: 1.0070x; 1.0070x over previous
import functools

import jax
import jax.numpy as jnp
from jax import lax
from jax.experimental import pallas as pl
from jax.experimental.pallas import tpu as pltpu
from jax.experimental.pallas import tpu_sc as plsc

NUM_SC_CORES = 2
NUM_SUBCORES = 16
NUM_WORKERS = NUM_SC_CORES * NUM_SUBCORES

BATCH = 16384
EMBED = 32
ROWS_PER_WORKER = BATCH // NUM_WORKERS
IDX_CHUNK = 128
NUM_CHUNKS = ROWS_PER_WORKER // IDX_CHUNK


def _sc_gather(user_table, item_table, user_ids, item_ids):
    mesh = plsc.VectorSubcoreMesh(core_axis_name="c", subcore_axis_name="s")

    @functools.partial(
        pl.kernel,
        mesh=mesh,
        compiler_params=pltpu.CompilerParams(use_tc_tiling_on_sc=False),
        out_type=(
            jax.ShapeDtypeStruct((BATCH, EMBED), jnp.float32),
            jax.ShapeDtypeStruct((BATCH, EMBED), jnp.float32),
        ),
        scratch_types=[
            pltpu.VMEM((ROWS_PER_WORKER,), jnp.int32),
            pltpu.VMEM((ROWS_PER_WORKER,), jnp.int32),
            pltpu.VMEM((ROWS_PER_WORKER, EMBED), jnp.float32),
            pltpu.VMEM((ROWS_PER_WORKER, EMBED), jnp.float32),
            pltpu.SemaphoreType.DMA,
            pltpu.SemaphoreType.DMA,
        ],
    )
    def k(ut_hbm, it_hbm, uid_hbm, iid_hbm, u_out, i_out,
          uidx_v, iidx_v, urows_v, irows_v, sem_u, sem_i):
        wid = lax.axis_index("s") * NUM_SC_CORES + lax.axis_index("c")
        base = wid * ROWS_PER_WORKER
        pltpu.sync_copy(uid_hbm.at[pl.ds(base, ROWS_PER_WORKER)], uidx_v)
        pltpu.sync_copy(iid_hbm.at[pl.ds(base, ROWS_PER_WORKER)], iidx_v)
        copies = []
        for j in range(NUM_CHUNKS):
            idx = pl.ds(j * IDX_CHUNK, IDX_CHUNK)
            copies.append(pltpu.async_copy(
                ut_hbm.at[uidx_v.at[idx]], urows_v.at[idx], sem_u))
            copies.append(pltpu.async_copy(
                it_hbm.at[iidx_v.at[idx]], irows_v.at[idx], sem_i))
        for c in copies:
            c.wait()
        pltpu.sync_copy(urows_v, u_out.at[pl.ds(base, ROWS_PER_WORKER)])
        pltpu.sync_copy(irows_v, i_out.at[pl.ds(base, ROWS_PER_WORKER)])

    return k(user_table, item_table, user_ids, item_ids)


def _tc_body(u_ref, i_ref, f_ref, wf_ref, bf_ref, w_ref, b_ref, out_ref):
    g_t = lax.dot_general(wf_ref[...], f_ref[...], (((1,), (0,)), ((), ())),
                          preferred_element_type=jnp.float32)
    inter = u_ref[...] * (i_ref[...] + g_t + bf_ref[...])
    out = lax.dot_general(w_ref[...], inter, (((1,), (0,)), ((), ())),
                          preferred_element_type=jnp.float32)
    out_ref[...] = out + b_ref[...]


def _tc_combine(u_t, i_t, f_t, Wf, bf2, W, b2):
    return pl.pallas_call(
        _tc_body,
        out_shape=jax.ShapeDtypeStruct((1, BATCH), jnp.float32),
    )(u_t, i_t, f_t, Wf, bf2, W, b2)


def kernel(user_ids, item_ids, features, user_table, item_table, Wf, bf, W, b):
    u, i = _sc_gather(user_table, item_table,
                      user_ids.astype(jnp.int32), item_ids.astype(jnp.int32))
    out = _tc_combine(u.T, i.T, features.T, Wf, bf.reshape(EMBED, 1), W,
                      b.reshape(1, 1))
    return out.reshape(BATCH)


# speedup vs baseline: 1.0097x; 1.0026x over previous
import functools

import jax
import jax.numpy as jnp
from jax import lax
from jax.experimental import pallas as pl
from jax.experimental.pallas import tpu as pltpu
from jax.experimental.pallas import tpu_sc as plsc

NUM_SC_CORES = 2
NUM_SUBCORES = 16
NUM_WORKERS = NUM_SC_CORES * NUM_SUBCORES

BATCH = 16384
EMBED = 32
ROWS_PER_WORKER = BATCH // NUM_WORKERS
IDX_CHUNK = 128
NUM_CHUNKS = ROWS_PER_WORKER // IDX_CHUNK


def _sc_gather_one(table, ids):
    mesh = plsc.VectorSubcoreMesh(core_axis_name="c", subcore_axis_name="s")

    @functools.partial(
        pl.kernel,
        mesh=mesh,
        compiler_params=pltpu.CompilerParams(use_tc_tiling_on_sc=False),
        out_type=jax.ShapeDtypeStruct((BATCH, EMBED), jnp.float32),
        scratch_types=[
            pltpu.VMEM((ROWS_PER_WORKER,), jnp.int32),
            pltpu.VMEM((ROWS_PER_WORKER, EMBED), jnp.float32),
            pltpu.SemaphoreType.DMA,
        ],
    )
    def k(t_hbm, id_hbm, out, idx_v, rows_v, sem):
        wid = lax.axis_index("s") * NUM_SC_CORES + lax.axis_index("c")
        base = wid * ROWS_PER_WORKER
        pltpu.sync_copy(id_hbm.at[pl.ds(base, ROWS_PER_WORKER)], idx_v)
        copies = []
        for j in range(NUM_CHUNKS):
            idx = pl.ds(j * IDX_CHUNK, IDX_CHUNK)
            copies.append(pltpu.async_copy(
                t_hbm.at[idx_v.at[idx]], rows_v.at[idx], sem))
        for c in copies:
            c.wait()
        pltpu.sync_copy(rows_v, out.at[pl.ds(base, ROWS_PER_WORKER)])

    return k(table, ids)


def _tc_body(u_ref, i_ref, f_ref, wf_ref, bf_ref, w_ref, b_ref, out_ref):
    g_t = lax.dot_general(wf_ref[...], f_ref[...], (((1,), (0,)), ((), ())),
                          preferred_element_type=jnp.float32)
    inter = u_ref[...] * (i_ref[...] + g_t + bf_ref[...])
    out = lax.dot_general(w_ref[...], inter, (((1,), (0,)), ((), ())),
                          preferred_element_type=jnp.float32)
    out_ref[...] = out + b_ref[...]


def _tc_combine(u_t, i_t, f_t, Wf, bf2, W, b2):
    return pl.pallas_call(
        _tc_body,
        out_shape=jax.ShapeDtypeStruct((1, BATCH), jnp.float32),
    )(u_t, i_t, f_t, Wf, bf2, W, b2)


def kernel(user_ids, item_ids, features, user_table, item_table, Wf, bf, W, b):
    u = _sc_gather_one(user_table, user_ids.astype(jnp.int32))
    i = _sc_gather_one(item_table, item_ids.astype(jnp.int32))
    out = _tc_combine(u.T, i.T, features.T, Wf, bf.reshape(EMBED, 1), W,
                      b.reshape(1, 1))
    return out.reshape(BATCH)
